# Initial kernel scaffold; baseline (speedup 1.0000x reference)
import functools

import jax
import jax.numpy as jnp
from jax import lax
from jax.experimental import pallas as pl
from jax.experimental.pallas import tpu as pltpu
from jax.experimental.pallas import tpu_sc as plsc

NCORES = 2
NTILES = 16
LANES = 16

D = 128
SLAB = 32
NSLAB = D // SLAB
NUNITS = 4 * NSLAB

CHUNK = 128
KROWS = 8
MACRO = CHUNK * KROWS


def _cdiv(a, b):
    return (a + b - 1) // b


def _splat(vec, e):
    return jnp.take_along_axis(vec, jnp.full((LANES,), e, jnp.int32), axis=0)


def _xt_body(x_ref, w_ref, b_ref, out_ref):
    x = x_ref[0]
    w = w_ref[0]
    b = b_ref[0]
    res = jnp.dot(x, w, preferred_element_type=jnp.float32) + b[None, :]
    res = jnp.maximum(res, 0.0)
    for p in range(NSLAB):
        out_ref[0, p] = res[:, SLAB * p:SLAB * (p + 1)]


def _make_xt_call(np_, bm):
    grid = (4, np_ // bm)
    return pl.pallas_call(
        _xt_body,
        grid=grid,
        in_specs=[
            pl.BlockSpec((1, bm, D), lambda r, i: (r, i, 0)),
            pl.BlockSpec((1, D, D), lambda r, i: (r, 0, 0)),
            pl.BlockSpec((1, D), lambda r, i: (r, 0)),
        ],
        out_specs=pl.BlockSpec((1, NSLAB, bm, SLAB), lambda r, i: (r, 0, i, 0)),
        out_shape=jax.ShapeDtypeStruct((4, NSLAB, np_, SLAB), jnp.float32),
        compiler_params=pltpu.CompilerParams(
            dimension_semantics=("parallel", "arbitrary"),
        ),
    )


def _sc_body(np_, ept, table, tgt_hbm, src_hbm, w_hbm,
             aggr_out, cnt_out,
             acc, cnt_acc,
             tgt_v, src_v, w_v, rows_v, zrow, z128, ones128,
             gsem, ssem, csem):
    c = lax.axis_index("c")
    t = lax.axis_index("s")

    stripe = np_ // NTILES
    nz = _cdiv(stripe, CHUNK)
    zlast = stripe - CHUNK
    rows_pt = ept // CHUNK
    nmacro = ept // MACRO

    def _init(i, carry):
        for g2 in range(SLAB // LANES):
            zrow[i, pl.ds(g2 * LANES, LANES)] = jnp.zeros((LANES,), jnp.float32)
        return carry
    lax.fori_loop(0, CHUNK, _init, 0)

    def _init1(i, carry):
        z128[pl.ds(i * LANES, LANES)] = jnp.zeros((LANES,), jnp.float32)
        ones128[pl.ds(i * LANES, LANES)] = jnp.ones((LANES,), jnp.float32)
        return carry
    lax.fori_loop(0, CHUNK // LANES, _init1, 0)

    def unit_body(i, carry):
        u = 2 * i + c
        r = u // NSLAB
        p = lax.rem(u, NSLAB)

        base = t * stripe

        def zloop(j, cc):
            lo = base + jnp.minimum(j * CHUNK, zlast)
            pltpu.sync_copy(zrow, acc.at[pl.ds(lo, CHUNK), :])
            return cc
        lax.fori_loop(0, nz, zloop, 0)

        @pl.when(p == 0)
        def _():
            def zc(j, cc):
                lo = base + jnp.minimum(j * CHUNK, zlast)
                pltpu.sync_copy(z128, cnt_acc.at[pl.ds(lo, CHUNK)])
                return cc
            lax.fori_loop(0, nz, zc, 0)

        plsc.subcore_barrier()

        off = u * np_

        def mloop(m, cc):
            ebase = t * ept + m * MACRO
            rowbase = t * rows_pt + m * KROWS
            pltpu.sync_copy(tgt_hbm.at[r, pl.ds(ebase, MACRO)], tgt_v)
            pltpu.sync_copy(w_hbm.at[r, pl.ds(ebase, MACRO)], w_v)
            pltpu.sync_copy(src_hbm.at[r, 0, pl.ds(rowbase, KROWS), :], src_v)

            def oloop(g, cc2):
                v = tgt_v[pl.ds(g * LANES, LANES)]
                tgt_v[pl.ds(g * LANES, LANES)] = v + off
                return cc2
            lax.fori_loop(0, MACRO // LANES, oloop, 0)

            cps = []
            for j in range(KROWS):
                cps.append(pltpu.async_copy(
                    table.at[tgt_v.at[pl.ds(j * CHUNK, CHUNK)]],
                    rows_v.at[pl.ds(j * CHUNK, CHUNK), :],
                    gsem))
            for cp in cps:
                cp.wait()

            def gloop(g, cc2):
                w_vec = w_v[pl.ds(g * LANES, LANES)]
                for e in range(LANES):
                    sp = _splat(w_vec, e)
                    q = g * LANES + e
                    a0 = rows_v[q, pl.ds(0, LANES)]
                    a1 = rows_v[q, pl.ds(LANES, LANES)]
                    rows_v[q, pl.ds(0, LANES)] = a0 * sp
                    rows_v[q, pl.ds(LANES, LANES)] = a1 * sp
                return cc2
            lax.fori_loop(0, MACRO // LANES, gloop, 0)

            cps = []
            for j in range(KROWS):
                cps.append(pltpu.async_copy(
                    rows_v.at[pl.ds(j * CHUNK, CHUNK), :],
                    acc.at[src_v.at[j]],
                    ssem, add=True))
            for cp in cps:
                cp.wait()

            @pl.when(p == 0)
            def _():
                ccps = []
                for j in range(KROWS):
                    ccps.append(pltpu.async_copy(
                        ones128, cnt_acc.at[src_v.at[j]], csem, add=True))
                for cp in ccps:
                    cp.wait()
            return cc
        lax.fori_loop(0, nmacro, mloop, 0)

        plsc.subcore_barrier()

        def wloop(j, cc):
            lo = base + jnp.minimum(j * CHUNK, zlast)
            pltpu.sync_copy(acc.at[pl.ds(lo, CHUNK), :],
                            aggr_out.at[u, pl.ds(lo, CHUNK), :])
            return cc
        lax.fori_loop(0, nz, wloop, 0)

        @pl.when(p == 0)
        def _():
            def wc(j, cc):
                lo = base + jnp.minimum(j * CHUNK, zlast)
                pltpu.sync_copy(cnt_acc.at[pl.ds(lo, CHUNK)],
                                cnt_out.at[r, pl.ds(lo, CHUNK)])
                return cc
            lax.fori_loop(0, nz, wc, 0)
        return carry

    lax.fori_loop(0, NUNITS // NCORES, unit_body, 0)


def _make_sc_call(np_, ept):
    mesh = plsc.VectorSubcoreMesh(
        core_axis_name="c", subcore_axis_name="s",
        num_cores=NCORES, num_subcores=NTILES)
    return pl.kernel(
        functools.partial(_sc_body, np_, ept),
        out_type=[
            jax.ShapeDtypeStruct((NUNITS, np_, SLAB), jnp.float32),
            jax.ShapeDtypeStruct((4, np_), jnp.float32),
        ],
        mesh=mesh,
        scratch_types=[
            pltpu.VMEM_SHARED((np_, SLAB), jnp.float32),
            pltpu.VMEM_SHARED((np_,), jnp.float32),
            pltpu.VMEM((MACRO,), jnp.int32),
            pltpu.VMEM((KROWS, CHUNK), jnp.int32),
            pltpu.VMEM((MACRO,), jnp.float32),
            pltpu.VMEM((MACRO, SLAB), jnp.float32),
            pltpu.VMEM((CHUNK, SLAB), jnp.float32),
            pltpu.VMEM((CHUNK,), jnp.float32),
            pltpu.VMEM((CHUNK,), jnp.float32),
            pltpu.SemaphoreType.DMA,
            pltpu.SemaphoreType.DMA,
            pltpu.SemaphoreType.DMA,
        ],
    )


def _final_body(ag_ref, cnt_ref, xn_ref, u1_ref, u2_ref,
                wo1_ref, wo2_ref, bo_ref, out_ref):
    xn = xn_ref[...]
    zn = jnp.dot(xn, u2_ref[...], preferred_element_type=jnp.float32)
    aggs = []
    scores = []
    for r in range(4):
        cnt = jnp.maximum(cnt_ref[r], 1.0)
        a = ag_ref[r] / cnt[:, None]
        z = jnp.dot(a, u1_ref[...], preferred_element_type=jnp.float32) + zn
        z = jnp.where(z >= 0.0, z, 0.01 * z)
        aggs.append(a)
        scores.append(jnp.exp(z))
    ssum = scores[0] + scores[1] + scores[2] + scores[3]
    comb = aggs[0] * (scores[0] / ssum)
    for r in range(1, 4):
        comb = comb + aggs[r] * (scores[r] / ssum)
    h = (jnp.dot(xn, wo1_ref[...], preferred_element_type=jnp.float32)
         + jnp.dot(comb, wo2_ref[...], preferred_element_type=jnp.float32)
         + bo_ref[...])
    h = jnp.maximum(h, 0.0)
    nrm = jnp.sqrt(jnp.sum(h * h, axis=1, keepdims=True))
    out_ref[...] = h / jnp.maximum(nrm, 1e-12)


def _make_final_call(np_, bn):
    grid = (np_ // bn,)
    return pl.pallas_call(
        _final_body,
        grid=grid,
        in_specs=[
            pl.BlockSpec((4, bn, D), lambda i: (0, i, 0)),
            pl.BlockSpec((4, bn), lambda i: (0, i)),
            pl.BlockSpec((bn, D), lambda i: (i, 0)),
            pl.BlockSpec((D, 1), lambda i: (0, 0)),
            pl.BlockSpec((D, 1), lambda i: (0, 0)),
            pl.BlockSpec((D, D), lambda i: (0, 0)),
            pl.BlockSpec((D, D), lambda i: (0, 0)),
            pl.BlockSpec((1, D), lambda i: (0, 0)),
        ],
        out_specs=pl.BlockSpec((bn, D), lambda i: (i, 0)),
        out_shape=jax.ShapeDtypeStruct((np_, D), jnp.float32),
        compiler_params=pltpu.CompilerParams(
            dimension_semantics=("arbitrary",),
        ),
    )


def kernel(x0, x1, x2, x3, e0, e1, e2, e3, w0, w1, w2, w3, x_node, num_node,
           W_a, b_a, W_p, b_p, W_t, b_t, W_c, b_c, u, W_out, b_out):
    n, d = x_node.shape
    assert d == D
    e = e0.shape[1]

    np_ = _cdiv(n, NTILES * 8) * NTILES * 8
    ept = _cdiv(e, NTILES * MACRO) * MACRO
    epad = NTILES * ept

    xs = jnp.stack([x0, x1, x2, x3])
    xs = jnp.pad(xs, ((0, 0), (0, np_ - n), (0, 0)))
    ws_mat = jnp.stack([W_a, W_p, W_t, W_c])
    bs = jnp.stack([b_a, b_p, b_t, b_c])

    pad = epad - e
    ar = jnp.arange(pad, dtype=jnp.int32)
    pad_src = n + ar % (np_ - n)
    pad_tgt = ar % 64
    srcs, tgts = [], []
    for er in (e0, e1, e2, e3):
        srcs.append(jnp.concatenate([er[0], pad_src]))
        tgts.append(jnp.concatenate([er[1], pad_tgt]))
    src_hbm = jnp.stack(srcs).reshape(4, 1, epad // CHUNK, CHUNK)
    tgt_hbm = jnp.stack(tgts)
    w_hbm = jnp.stack([
        jnp.concatenate([wr, jnp.zeros((pad,), jnp.float32)])
        for wr in (w0, w1, w2, w3)
    ])

    tables = _make_xt_call(np_, 3128)(xs, ws_mat, bs)
    table_flat = tables.reshape(4 * NSLAB * np_, SLAB)

    aggr16, cnts = _make_sc_call(np_, ept)(
        table_flat, tgt_hbm, src_hbm, w_hbm)

    aggr4 = aggr16.reshape(4, NSLAB, np_, SLAB).transpose(0, 2, 1, 3)
    aggr4 = aggr4.reshape(4, np_, D)
    xn = jnp.pad(x_node, ((0, np_ - n), (0, 0)))
    u1 = u[:D]
    u2 = u[D:]
    wo1 = W_out[:D]
    wo2 = W_out[D:]
    out = _make_final_call(np_, 3128)(
        aggr4, cnts, xn, u1, u2, wo1, wo2, b_out.reshape(1, D))
    return out[:n]


# speedup vs baseline: 2.6703x; 2.6703x over previous
import functools

import jax
import jax.numpy as jnp
from jax import lax
from jax.experimental import pallas as pl
from jax.experimental.pallas import tpu as pltpu
from jax.experimental.pallas import tpu_sc as plsc

NCORES = 2
NTILES = 16
LANES = 16

D = 128
SLAB = 32
NSLAB = D // SLAB
NUNITS = 4 * NSLAB

CHUNK = 128
KROWS = 4
MACRO = CHUNK * KROWS


def _cdiv(a, b):
    return (a + b - 1) // b


def _splat(vec, e):
    return jnp.take_along_axis(vec, jnp.full((LANES,), e, jnp.int32), axis=0)


def _xt_body(x_ref, w_ref, b_ref, out_ref):
    x = x_ref[0]
    w = w_ref[0]
    b = b_ref[pl.program_id(0)]
    res = jnp.dot(x, w, preferred_element_type=jnp.float32) + b[None, :]
    res = jnp.maximum(res, 0.0)
    for p in range(NSLAB):
        out_ref[0, p] = res[:, SLAB * p:SLAB * (p + 1)]


def _make_xt_call(np_, bm):
    grid = (4, np_ // bm)
    return pl.pallas_call(
        _xt_body,
        grid=grid,
        in_specs=[
            pl.BlockSpec((1, bm, D), lambda r, i: (r, i, 0)),
            pl.BlockSpec((1, D, D), lambda r, i: (r, 0, 0)),
            pl.BlockSpec((4, D), lambda r, i: (0, 0)),
        ],
        out_specs=pl.BlockSpec((1, NSLAB, bm, SLAB), lambda r, i: (r, 0, i, 0)),
        out_shape=jax.ShapeDtypeStruct((4, NSLAB, np_, SLAB), jnp.float32),
        compiler_params=pltpu.CompilerParams(
            dimension_semantics=("parallel", "arbitrary"),
        ),
    )


def _sc_body(np_, ept, table, tgt_hbm, src_hbm, w_hbm,
             aggr_out, cnt_out,
             acc, cnt_acc,
             tgt_v, src_v, w_v, rows_v, zrow, z128, ones128,
             gsem, ssem, csem):
    c = lax.axis_index("c")
    t = lax.axis_index("s")

    stripe = np_ // NTILES
    nz = _cdiv(stripe, CHUNK)
    zlast = stripe - CHUNK
    rows_pt = ept // CHUNK
    nmacro = ept // MACRO

    def _init(i, carry):
        for g2 in range(SLAB // LANES):
            zrow[i, pl.ds(g2 * LANES, LANES)] = jnp.zeros((LANES,), jnp.float32)
        return carry
    lax.fori_loop(0, CHUNK, _init, 0)

    def _init1(i, carry):
        z128[pl.ds(i * LANES, LANES)] = jnp.zeros((LANES,), jnp.float32)
        ones128[pl.ds(i * LANES, LANES)] = jnp.ones((LANES,), jnp.float32)
        return carry
    lax.fori_loop(0, CHUNK // LANES, _init1, 0)

    def unit_body(i, carry):
        u = 2 * i + c
        r = u // NSLAB
        p = lax.rem(u, NSLAB)

        base = t * stripe

        def zloop(j, cc):
            lo = base + jnp.minimum(j * CHUNK, zlast)
            pltpu.sync_copy(zrow, acc.at[pl.ds(lo, CHUNK), :])
            return cc
        lax.fori_loop(0, nz, zloop, 0)

        @pl.when(p == 0)
        def _():
            def zc(j, cc):
                lo = base + jnp.minimum(j * CHUNK, zlast)
                pltpu.sync_copy(z128, cnt_acc.at[pl.ds(lo, CHUNK)])
                return cc
            lax.fori_loop(0, nz, zc, 0)

        plsc.subcore_barrier()

        off = u * np_

        def mloop(m, cc):
            ebase = t * ept + m * MACRO
            rowbase = t * rows_pt + m * KROWS
            pltpu.sync_copy(tgt_hbm.at[r, pl.ds(ebase, MACRO)], tgt_v)
            pltpu.sync_copy(w_hbm.at[r, pl.ds(ebase, MACRO)], w_v)
            pltpu.sync_copy(src_hbm.at[r, 0, pl.ds(rowbase, KROWS), :], src_v)

            def oloop(g, cc2):
                v = tgt_v[pl.ds(g * LANES, LANES)]
                tgt_v[pl.ds(g * LANES, LANES)] = v + off
                return cc2
            lax.fori_loop(0, MACRO // LANES, oloop, 0)

            cps = []
            for j in range(KROWS):
                cps.append(pltpu.async_copy(
                    table.at[tgt_v.at[pl.ds(j * CHUNK, CHUNK)]],
                    rows_v.at[pl.ds(j * CHUNK, CHUNK), :],
                    gsem))
            for cp in cps:
                cp.wait()

            def gloop(g, cc2):
                w_vec = w_v[pl.ds(g * LANES, LANES)]
                for e in range(LANES):
                    sp = _splat(w_vec, e)
                    q = g * LANES + e
                    a0 = rows_v[q, pl.ds(0, LANES)]
                    a1 = rows_v[q, pl.ds(LANES, LANES)]
                    rows_v[q, pl.ds(0, LANES)] = a0 * sp
                    rows_v[q, pl.ds(LANES, LANES)] = a1 * sp
                return cc2
            lax.fori_loop(0, MACRO // LANES, gloop, 0)

            cps = []
            for j in range(KROWS):
                cps.append(pltpu.async_copy(
                    rows_v.at[pl.ds(j * CHUNK, CHUNK), :],
                    acc.at[src_v.at[j]],
                    ssem, add=True))
            for cp in cps:
                cp.wait()

            @pl.when(p == 0)
            def _():
                ccps = []
                for j in range(KROWS):
                    ccps.append(pltpu.async_copy(
                        ones128, cnt_acc.at[src_v.at[j]], csem, add=True))
                for cp in ccps:
                    cp.wait()
            return cc
        lax.fori_loop(0, nmacro, mloop, 0)

        plsc.subcore_barrier()

        def wloop(j, cc):
            lo = base + jnp.minimum(j * CHUNK, zlast)
            pltpu.sync_copy(acc.at[pl.ds(lo, CHUNK), :],
                            aggr_out.at[u, pl.ds(lo, CHUNK), :])
            return cc
        lax.fori_loop(0, nz, wloop, 0)

        @pl.when(p == 0)
        def _():
            def wc(j, cc):
                lo = base + jnp.minimum(j * CHUNK, zlast)
                pltpu.sync_copy(cnt_acc.at[pl.ds(lo, CHUNK)],
                                cnt_out.at[r, pl.ds(lo, CHUNK)])
                return cc
            lax.fori_loop(0, nz, wc, 0)
        return carry

    lax.fori_loop(0, NUNITS // NCORES, unit_body, 0)


def _make_sc_call(np_, ept):
    mesh = plsc.VectorSubcoreMesh(
        core_axis_name="c", subcore_axis_name="s",
        num_cores=NCORES, num_subcores=NTILES)
    return pl.kernel(
        functools.partial(_sc_body, np_, ept),
        out_type=[
            jax.ShapeDtypeStruct((NUNITS, np_, SLAB), jnp.float32),
            jax.ShapeDtypeStruct((4, np_), jnp.float32),
        ],
        mesh=mesh,
        compiler_params=pltpu.CompilerParams(use_tc_tiling_on_sc=False),
        scratch_types=[
            pltpu.VMEM_SHARED((np_, SLAB), jnp.float32),
            pltpu.VMEM_SHARED((np_,), jnp.float32),
            pltpu.VMEM((MACRO,), jnp.int32),
            pltpu.VMEM((KROWS, CHUNK), jnp.int32),
            pltpu.VMEM((MACRO,), jnp.float32),
            pltpu.VMEM((MACRO, SLAB), jnp.float32),
            pltpu.VMEM((CHUNK, SLAB), jnp.float32),
            pltpu.VMEM((CHUNK,), jnp.float32),
            pltpu.VMEM((CHUNK,), jnp.float32),
            pltpu.SemaphoreType.DMA,
            pltpu.SemaphoreType.DMA,
            pltpu.SemaphoreType.DMA,
        ],
    )


def _final_body(ag_ref, cnt_ref, xn_ref, u1_ref, u2_ref,
                wo1_ref, wo2_ref, bo_ref, out_ref):
    xn = xn_ref[...]
    zn = jnp.dot(xn, u2_ref[...], preferred_element_type=jnp.float32)
    aggs = []
    scores = []
    for r in range(4):
        cnt = jnp.maximum(cnt_ref[r], 1.0)
        a = ag_ref[r] / cnt[:, None]
        z = jnp.dot(a, u1_ref[...], preferred_element_type=jnp.float32) + zn
        z = jnp.where(z >= 0.0, z, 0.01 * z)
        aggs.append(a)
        scores.append(jnp.exp(z))
    ssum = scores[0] + scores[1] + scores[2] + scores[3]
    comb = aggs[0] * (scores[0] / ssum)
    for r in range(1, 4):
        comb = comb + aggs[r] * (scores[r] / ssum)
    h = (jnp.dot(xn, wo1_ref[...], preferred_element_type=jnp.float32)
         + jnp.dot(comb, wo2_ref[...], preferred_element_type=jnp.float32)
         + bo_ref[...])
    h = jnp.maximum(h, 0.0)
    nrm = jnp.sqrt(jnp.sum(h * h, axis=1, keepdims=True))
    out_ref[...] = h / jnp.maximum(nrm, 1e-12)


def _make_final_call(np_, bn):
    grid = (np_ // bn,)
    return pl.pallas_call(
        _final_body,
        grid=grid,
        in_specs=[
            pl.BlockSpec((4, bn, D), lambda i: (0, i, 0)),
            pl.BlockSpec((4, bn), lambda i: (0, i)),
            pl.BlockSpec((bn, D), lambda i: (i, 0)),
            pl.BlockSpec((D, 1), lambda i: (0, 0)),
            pl.BlockSpec((D, 1), lambda i: (0, 0)),
            pl.BlockSpec((D, D), lambda i: (0, 0)),
            pl.BlockSpec((D, D), lambda i: (0, 0)),
            pl.BlockSpec((1, D), lambda i: (0, 0)),
        ],
        out_specs=pl.BlockSpec((bn, D), lambda i: (i, 0)),
        out_shape=jax.ShapeDtypeStruct((np_, D), jnp.float32),
        compiler_params=pltpu.CompilerParams(
            dimension_semantics=("arbitrary",),
        ),
    )


def kernel(x0, x1, x2, x3, e0, e1, e2, e3, w0, w1, w2, w3, x_node, num_node,
           W_a, b_a, W_p, b_p, W_t, b_t, W_c, b_c, u, W_out, b_out):
    n, d = x_node.shape
    assert d == D
    e = e0.shape[1]

    np_ = _cdiv(n, NTILES * 128) * NTILES * 128
    ept = _cdiv(e, NTILES * MACRO) * MACRO
    epad = NTILES * ept

    xs = jnp.stack([x0, x1, x2, x3])
    xs = jnp.pad(xs, ((0, 0), (0, np_ - n), (0, 0)))
    ws_mat = jnp.stack([W_a, W_p, W_t, W_c])
    bs = jnp.stack([b_a, b_p, b_t, b_c])

    pad = epad - e
    ar = jnp.arange(pad, dtype=jnp.int32)
    pad_src = n + ar % (np_ - n)
    pad_tgt = ar % 64
    srcs, tgts = [], []
    for er in (e0, e1, e2, e3):
        srcs.append(jnp.concatenate([er[0], pad_src]))
        tgts.append(jnp.concatenate([er[1], pad_tgt]))
    src_hbm = jnp.stack(srcs).reshape(4, 1, epad // CHUNK, CHUNK)
    tgt_hbm = jnp.stack(tgts)
    w_hbm = jnp.stack([
        jnp.concatenate([wr, jnp.zeros((pad,), jnp.float32)])
        for wr in (w0, w1, w2, w3)
    ])

    tables = _make_xt_call(np_, np_ // 16)(xs, ws_mat, bs)
    table_flat = tables.reshape(4 * NSLAB * np_, SLAB)

    aggr16, cnts = _make_sc_call(np_, ept)(
        table_flat, tgt_hbm, src_hbm, w_hbm)

    aggr4 = aggr16.reshape(4, NSLAB, np_, SLAB).transpose(0, 2, 1, 3)
    aggr4 = aggr4.reshape(4, np_, D)
    xn = jnp.pad(x_node, ((0, np_ - n), (0, 0)))
    u1 = u[:D]
    u2 = u[D:]
    wo1 = W_out[:D]
    wo2 = W_out[D:]
    out = _make_final_call(np_, np_ // 16)(
        aggr4, cnts, xn, u1, u2, wo1, wo2, b_out.reshape(1, D))
    return out[:n]


# speedup vs baseline: 4.4979x; 1.6844x over previous
import functools

import jax
import jax.numpy as jnp
from jax import lax
from jax.experimental import pallas as pl
from jax.experimental.pallas import tpu as pltpu
from jax.experimental.pallas import tpu_sc as plsc

NCORES = 2
NTILES = 16
LANES = 16

D = 128
SLAB = 32
NSLAB = D // SLAB
NUNITS = 4 * NSLAB

CHUNK = 128
KROWS = 2
MACRO = CHUNK * KROWS


def _cdiv(a, b):
    return (a + b - 1) // b


def _splat(vec, e):
    return jnp.take_along_axis(vec, jnp.full((LANES,), e, jnp.int32), axis=0)


def _xt_body(x_ref, w_ref, b_ref, out_ref):
    x = x_ref[0]
    w = w_ref[0]
    b = b_ref[pl.program_id(0)]
    res = jnp.dot(x, w, preferred_element_type=jnp.float32) + b[None, :]
    res = jnp.maximum(res, 0.0)
    for p in range(NSLAB):
        out_ref[0, p] = res[:, SLAB * p:SLAB * (p + 1)]


def _make_xt_call(np_, bm):
    grid = (4, np_ // bm)
    return pl.pallas_call(
        _xt_body,
        grid=grid,
        in_specs=[
            pl.BlockSpec((1, bm, D), lambda r, i: (r, i, 0)),
            pl.BlockSpec((1, D, D), lambda r, i: (r, 0, 0)),
            pl.BlockSpec((4, D), lambda r, i: (0, 0)),
        ],
        out_specs=pl.BlockSpec((1, NSLAB, bm, SLAB), lambda r, i: (r, 0, i, 0)),
        out_shape=jax.ShapeDtypeStruct((4, NSLAB, np_, SLAB), jnp.float32),
        compiler_params=pltpu.CompilerParams(
            dimension_semantics=("parallel", "arbitrary"),
        ),
    )


def _sc_body(np_, acc_n, ept, table, tgt_hbm, src_hbm, w_hbm,
             aggr_out, cnt_out,
             acc, cnt_acc,
             tgt0, tgt1, src0, src1, w0_v, w1_v, rows0, rows1, scat0, scat1,
             zrow, z128, ones128,
             isem, gsem, ssem, csem, zsem):
    c = lax.axis_index("c")
    t = lax.axis_index("s")

    stripe = acc_n // NTILES
    nz = _cdiv(stripe, CHUNK)
    zlast = stripe - CHUNK
    rows_pt = ept // CHUNK
    nmacro = ept // MACRO

    bufs = ((tgt0, src0, w0_v, rows0, scat0),
            (tgt1, src1, w1_v, rows1, scat1))

    def _init(i, carry):
        for g2 in range(SLAB // LANES):
            zrow[i, pl.ds(g2 * LANES, LANES)] = jnp.zeros((LANES,), jnp.float32)
        return carry
    lax.fori_loop(0, CHUNK, _init, 0)

    def _init1(i, carry):
        z128[pl.ds(i * LANES, LANES)] = jnp.zeros((LANES,), jnp.float32)
        ones128[pl.ds(i * LANES, LANES)] = jnp.ones((LANES,), jnp.float32)
        return carry
    lax.fori_loop(0, CHUNK // LANES, _init1, 0)

    def unit_body(i, carry):
        u = 2 * i + c
        r = u // NSLAB
        p = lax.rem(u, NSLAB)
        base = t * stripe
        off = u * np_

        def idx_descs(m, bi):
            tgt_b, src_b, w_b = bufs[bi][0], bufs[bi][1], bufs[bi][2]
            ebase = t * ept + m * MACRO
            rowbase = t * rows_pt + m * KROWS
            return (
                pltpu.make_async_copy(tgt_hbm.at[r, pl.ds(ebase, MACRO)], tgt_b, isem),
                pltpu.make_async_copy(w_hbm.at[r, pl.ds(ebase, MACRO)], w_b, isem),
                pltpu.make_async_copy(src_hbm.at[r, 0, pl.ds(rowbase, KROWS), :], src_b, isem),
            )

        def gather_descs(bi):
            tgt_b, rows_b = bufs[bi][0], bufs[bi][3]
            return tuple(
                pltpu.make_async_copy(
                    table.at[tgt_b.at[pl.ds(j * CHUNK, CHUNK)]],
                    rows_b.at[pl.ds(j * CHUNK, CHUNK), :], gsem)
                for j in range(KROWS))

        def scat_descs(bi):
            rows_b, scat_b = bufs[bi][3], bufs[bi][4]
            return tuple(
                pltpu.make_async_copy(
                    rows_b.at[pl.ds(j * CHUNK, CHUNK), :],
                    acc.at[scat_b.at[j]], ssem)
                for j in range(KROWS))

        def cnt_descs(bi):
            scat_b = bufs[bi][4]
            return tuple(
                pltpu.make_async_copy(ones128, cnt_acc.at[scat_b.at[j]], csem)
                for j in range(KROWS))

        def bias_tgt(bi):
            tgt_b = bufs[bi][0]

            def oloop(g, cc2):
                v = tgt_b[pl.ds(g * LANES, LANES)]
                tgt_b[pl.ds(g * LANES, LANES)] = v + off
                return cc2
            lax.fori_loop(0, MACRO // LANES, oloop, 0)

        def zero_desc(j):
            lo = base + jnp.minimum(j * CHUNK, zlast)
            return pltpu.make_async_copy(zrow, acc.at[pl.ds(lo, CHUNK), :], zsem)

        def zero_cnt_desc(j):
            lo = base + jnp.minimum(j * CHUNK, zlast)
            return pltpu.make_async_copy(z128, cnt_acc.at[pl.ds(lo, CHUNK)], zsem)

        lax.fori_loop(0, nz, lambda j, cc: (zero_desc(j).start(), cc)[1], 0)

        @pl.when(p == 0)
        def _():
            lax.fori_loop(0, nz, lambda j, cc: (zero_cnt_desc(j).start(), cc)[1], 0)

        lax.fori_loop(0, nz, lambda j, cc: (zero_desc(j).wait(), cc)[1], 0)

        @pl.when(p == 0)
        def _():
            lax.fori_loop(0, nz, lambda j, cc: (zero_cnt_desc(j).wait(), cc)[1], 0)

        plsc.subcore_barrier()

        for d in idx_descs(0, 0):
            d.start()
        for d in idx_descs(0, 0):
            d.wait()
        for d in idx_descs(1, 1):
            d.start()
        bias_tgt(0)
        for d in gather_descs(0):
            d.start()

        def process(m, cur, oth):
            tgt_c, src_c, w_c, rows_c, scat_c = bufs[cur]

            for d in gather_descs(cur):
                d.wait()

            @pl.when(m >= 1)
            def _():
                for d in scat_descs(oth):
                    d.wait()

                @pl.when(p == 0)
                def _():
                    for d in cnt_descs(oth):
                        d.wait()

            @pl.when(m + 1 < nmacro)
            def _():
                for d in idx_descs(m + 1, oth):
                    d.wait()
                bias_tgt(oth)
                for d in gather_descs(oth):
                    d.start()

            def gloop(g, cc2):
                w_vec = w_c[pl.ds(g * LANES, LANES)]
                for e in range(LANES):
                    sp = _splat(w_vec, e)
                    q = g * LANES + e
                    a0 = rows_c[q, pl.ds(0, LANES)]
                    a1 = rows_c[q, pl.ds(LANES, LANES)]
                    rows_c[q, pl.ds(0, LANES)] = a0 * sp
                    rows_c[q, pl.ds(LANES, LANES)] = a1 * sp
                return cc2
            lax.fori_loop(0, MACRO // LANES, gloop, 0)
            for j in range(KROWS):
                for g2 in range(CHUNK // LANES):
                    scat_c[j, pl.ds(g2 * LANES, LANES)] = \
                        src_c[j, pl.ds(g2 * LANES, LANES)]

            for d in scat_descs(cur):
                d.start(add=True)

            @pl.when(p == 0)
            def _():
                for d in cnt_descs(cur):
                    d.start(add=True)

            @pl.when(m + 2 < nmacro)
            def _():
                for d in idx_descs(m + 2, cur):
                    d.start()

        def pair(mm, cc):
            m0 = 2 * mm
            process(m0, 0, 1)
            process(m0 + 1, 1, 0)
            return cc
        lax.fori_loop(0, nmacro // 2, pair, 0)

        for d in scat_descs(1):
            d.wait()

        @pl.when(p == 0)
        def _():
            for d in cnt_descs(1):
                d.wait()

        plsc.subcore_barrier()

        def wout_desc(j):
            lo = base + jnp.minimum(j * CHUNK, zlast)
            return pltpu.make_async_copy(
                acc.at[pl.ds(lo, CHUNK), :],
                aggr_out.at[u, pl.ds(lo, CHUNK), :], zsem)

        def wout_cnt_desc(j):
            lo = base + jnp.minimum(j * CHUNK, zlast)
            return pltpu.make_async_copy(
                cnt_acc.at[pl.ds(lo, CHUNK)],
                cnt_out.at[r, pl.ds(lo, CHUNK)], zsem)

        lax.fori_loop(0, nz, lambda j, cc: (wout_desc(j).start(), cc)[1], 0)

        @pl.when(p == 0)
        def _():
            lax.fori_loop(0, nz, lambda j, cc: (wout_cnt_desc(j).start(), cc)[1], 0)

        lax.fori_loop(0, nz, lambda j, cc: (wout_desc(j).wait(), cc)[1], 0)

        @pl.when(p == 0)
        def _():
            lax.fori_loop(0, nz, lambda j, cc: (wout_cnt_desc(j).wait(), cc)[1], 0)
        return carry

    lax.fori_loop(0, NUNITS // NCORES, unit_body, 0)


def _make_sc_call(np_, acc_n, ept):
    mesh = plsc.VectorSubcoreMesh(
        core_axis_name="c", subcore_axis_name="s",
        num_cores=NCORES, num_subcores=NTILES)
    return pl.kernel(
        functools.partial(_sc_body, np_, acc_n, ept),
        out_type=[
            jax.ShapeDtypeStruct((NUNITS, np_, SLAB), jnp.float32),
            jax.ShapeDtypeStruct((4, np_), jnp.float32),
        ],
        mesh=mesh,
        compiler_params=pltpu.CompilerParams(use_tc_tiling_on_sc=False),
        scratch_types=[
            pltpu.VMEM_SHARED((acc_n, SLAB), jnp.float32),
            pltpu.VMEM_SHARED((acc_n,), jnp.float32),
            pltpu.VMEM((MACRO,), jnp.int32),
            pltpu.VMEM((MACRO,), jnp.int32),
            pltpu.VMEM((KROWS, CHUNK), jnp.int32),
            pltpu.VMEM((KROWS, CHUNK), jnp.int32),
            pltpu.VMEM((MACRO,), jnp.float32),
            pltpu.VMEM((MACRO,), jnp.float32),
            pltpu.VMEM((MACRO, SLAB), jnp.float32),
            pltpu.VMEM((MACRO, SLAB), jnp.float32),
            pltpu.VMEM((KROWS, CHUNK), jnp.int32),
            pltpu.VMEM((KROWS, CHUNK), jnp.int32),
            pltpu.VMEM((CHUNK, SLAB), jnp.float32),
            pltpu.VMEM((CHUNK,), jnp.float32),
            pltpu.VMEM((CHUNK,), jnp.float32),
            pltpu.SemaphoreType.DMA,
            pltpu.SemaphoreType.DMA,
            pltpu.SemaphoreType.DMA,
            pltpu.SemaphoreType.DMA,
            pltpu.SemaphoreType.DMA,
        ],
    )


def _final_body(ag_ref, cnt_ref, xn_ref, u1_ref, u2_ref,
                wo1_ref, wo2_ref, bo_ref, out_ref):
    xn = xn_ref[...]
    zn = jnp.dot(xn, u2_ref[...], preferred_element_type=jnp.float32)
    aggs = []
    scores = []
    for r in range(4):
        cnt = jnp.maximum(cnt_ref[r], 1.0)
        a = ag_ref[r] / cnt[:, None]
        z = jnp.dot(a, u1_ref[...], preferred_element_type=jnp.float32) + zn
        z = jnp.where(z >= 0.0, z, 0.01 * z)
        aggs.append(a)
        scores.append(jnp.exp(z))
    ssum = scores[0] + scores[1] + scores[2] + scores[3]
    comb = aggs[0] * (scores[0] / ssum)
    for r in range(1, 4):
        comb = comb + aggs[r] * (scores[r] / ssum)
    h = (jnp.dot(xn, wo1_ref[...], preferred_element_type=jnp.float32)
         + jnp.dot(comb, wo2_ref[...], preferred_element_type=jnp.float32)
         + bo_ref[...])
    h = jnp.maximum(h, 0.0)
    nrm = jnp.sqrt(jnp.sum(h * h, axis=1, keepdims=True))
    out_ref[...] = h / jnp.maximum(nrm, 1e-12)


def _make_final_call(np_, bn):
    grid = (np_ // bn,)
    return pl.pallas_call(
        _final_body,
        grid=grid,
        in_specs=[
            pl.BlockSpec((4, bn, D), lambda i: (0, i, 0)),
            pl.BlockSpec((4, bn), lambda i: (0, i)),
            pl.BlockSpec((bn, D), lambda i: (i, 0)),
            pl.BlockSpec((D, 1), lambda i: (0, 0)),
            pl.BlockSpec((D, 1), lambda i: (0, 0)),
            pl.BlockSpec((D, D), lambda i: (0, 0)),
            pl.BlockSpec((D, D), lambda i: (0, 0)),
            pl.BlockSpec((1, D), lambda i: (0, 0)),
        ],
        out_specs=pl.BlockSpec((bn, D), lambda i: (i, 0)),
        out_shape=jax.ShapeDtypeStruct((np_, D), jnp.float32),
        compiler_params=pltpu.CompilerParams(
            dimension_semantics=("arbitrary",),
        ),
    )


def kernel(x0, x1, x2, x3, e0, e1, e2, e3, w0, w1, w2, w3, x_node, num_node,
           W_a, b_a, W_p, b_p, W_t, b_t, W_c, b_c, u, W_out, b_out):
    n, d = x_node.shape
    assert d == D
    e = e0.shape[1]

    np_ = _cdiv(n, NTILES * 128) * NTILES * 128
    acc_n = _cdiv(n, NTILES * 8) * NTILES * 8
    ept = _cdiv(e, NTILES * 2 * MACRO) * 2 * MACRO
    epad = NTILES * ept

    xs = jnp.stack([x0, x1, x2, x3])
    xs = jnp.pad(xs, ((0, 0), (0, np_ - n), (0, 0)))
    ws_mat = jnp.stack([W_a, W_p, W_t, W_c])
    bs = jnp.stack([b_a, b_p, b_t, b_c])

    pad = epad - e
    ar = jnp.arange(pad, dtype=jnp.int32)
    pad_src = n + ar % (acc_n - n)
    pad_tgt = ar % 64
    srcs, tgts = [], []
    for er in (e0, e1, e2, e3):
        srcs.append(jnp.concatenate([er[0], pad_src]))
        tgts.append(jnp.concatenate([er[1], pad_tgt]))
    src_hbm = jnp.stack(srcs).reshape(4, 1, epad // CHUNK, CHUNK)
    tgt_hbm = jnp.stack(tgts)
    w_hbm = jnp.stack([
        jnp.concatenate([wr, jnp.zeros((pad,), jnp.float32)])
        for wr in (w0, w1, w2, w3)
    ])

    tables = _make_xt_call(np_, np_ // 16)(xs, ws_mat, bs)
    table_flat = tables.reshape(4 * NSLAB * np_, SLAB)

    aggr16, cnts = _make_sc_call(np_, acc_n, ept)(
        table_flat, tgt_hbm, src_hbm, w_hbm)

    aggr4 = aggr16.reshape(4, NSLAB, np_, SLAB).transpose(0, 2, 1, 3)
    aggr4 = aggr4.reshape(4, np_, D)
    xn = jnp.pad(x_node, ((0, np_ - n), (0, 0)))
    u1 = u[:D]
    u2 = u[D:]
    wo1 = W_out[:D]
    wo2 = W_out[D:]
    out = _make_final_call(np_, np_ // 16)(
        aggr4, cnts, xn, u1, u2, wo1, wo2, b_out.reshape(1, D))
    return out[:n]


# speedup vs baseline: 6.3674x; 1.4156x over previous
import functools

import jax
import jax.numpy as jnp
from jax import lax
from jax.experimental import pallas as pl
from jax.experimental.pallas import tpu as pltpu
from jax.experimental.pallas import tpu_sc as plsc

NCORES = 2
NTILES = 16
LANES = 16

D = 128
SLAB = 32
NSLAB = D // SLAB
NUNITS = 4 * NSLAB

CHUNK = 128
KROWS = 2
MACRO = CHUNK * KROWS


def _cdiv(a, b):
    return (a + b - 1) // b


def _splat(vec, e):
    return jnp.take_along_axis(vec, jnp.full((LANES,), e, jnp.int32), axis=0)


def _xt_body(x0r, x1r, x2r, x3r, w_ref, b_ref, out_ref):
    for k, xr in enumerate((x0r, x1r, x2r, x3r)):
        res = jnp.dot(xr[...], w_ref[k], preferred_element_type=jnp.float32)
        out_ref[k] = jnp.maximum(res + b_ref[k][None, :], 0.0)


def _make_xt_call(np_, bm):
    grid = (np_ // bm,)
    return pl.pallas_call(
        _xt_body,
        grid=grid,
        in_specs=[pl.BlockSpec((bm, D), lambda i: (i, 0))] * 4 + [
            pl.BlockSpec((4, D, D), lambda i: (0, 0, 0)),
            pl.BlockSpec((4, D), lambda i: (0, 0)),
        ],
        out_specs=pl.BlockSpec((4, bm, D), lambda i: (0, i, 0)),
        out_shape=jax.ShapeDtypeStruct((4, np_, D), jnp.float32),
        compiler_params=pltpu.CompilerParams(
            dimension_semantics=("arbitrary",),
        ),
    )


def _sc_body(np_, acc_n, ept, table, tgt_hbm, src_hbm, w_hbm,
             aggr_out, cnt_out,
             acc, cnt_acc,
             tgt0, tgt1, src0, src1, w0_v, w1_v, rows0, rows1, scat0, scat1,
             zrow, z128, ones128,
             isem, gsem, ssem, csem, zsem):
    c = lax.axis_index("c")
    t = lax.axis_index("s")

    stripe = acc_n // NTILES
    nz = _cdiv(stripe, CHUNK)
    zlast = stripe - CHUNK
    rows_pt = ept // CHUNK
    nmacro = ept // MACRO

    bufs = ((tgt0, src0, w0_v, rows0, scat0),
            (tgt1, src1, w1_v, rows1, scat1))

    def _init(i, carry):
        for g2 in range(SLAB // LANES):
            zrow[i, pl.ds(g2 * LANES, LANES)] = jnp.zeros((LANES,), jnp.float32)
        return carry
    lax.fori_loop(0, CHUNK, _init, 0)

    def _init1(i, carry):
        z128[pl.ds(i * LANES, LANES)] = jnp.zeros((LANES,), jnp.float32)
        ones128[pl.ds(i * LANES, LANES)] = jnp.ones((LANES,), jnp.float32)
        return carry
    lax.fori_loop(0, CHUNK // LANES, _init1, 0)

    def unit_body(i, carry):
        u = 2 * i + c
        r = u // NSLAB
        p = lax.rem(u, NSLAB)
        base = t * stripe
        off = 4 * r * np_ + p

        def idx_descs(m, bi):
            tgt_b, src_b, w_b = bufs[bi][0], bufs[bi][1], bufs[bi][2]
            ebase = t * ept + m * MACRO
            rowbase = t * rows_pt + m * KROWS
            return (
                pltpu.make_async_copy(tgt_hbm.at[r, pl.ds(ebase, MACRO)], tgt_b, isem),
                pltpu.make_async_copy(w_hbm.at[r, pl.ds(ebase, MACRO)], w_b, isem),
                pltpu.make_async_copy(src_hbm.at[r, 0, pl.ds(rowbase, KROWS), :], src_b, isem),
            )

        def gather_descs(bi):
            tgt_b, rows_b = bufs[bi][0], bufs[bi][3]
            return tuple(
                pltpu.make_async_copy(
                    table.at[tgt_b.at[pl.ds(j * CHUNK, CHUNK)]],
                    rows_b.at[pl.ds(j * CHUNK, CHUNK), :], gsem)
                for j in range(KROWS))

        def scat_descs(bi):
            rows_b, scat_b = bufs[bi][3], bufs[bi][4]
            return tuple(
                pltpu.make_async_copy(
                    rows_b.at[pl.ds(j * CHUNK, CHUNK), :],
                    acc.at[scat_b.at[j]], ssem)
                for j in range(KROWS))

        def cnt_descs(bi):
            scat_b = bufs[bi][4]
            return tuple(
                pltpu.make_async_copy(ones128, cnt_acc.at[scat_b.at[j]], csem)
                for j in range(KROWS))

        def bias_tgt(bi):
            tgt_b = bufs[bi][0]

            def oloop(g, cc2):
                v = tgt_b[pl.ds(g * LANES, LANES)]
                tgt_b[pl.ds(g * LANES, LANES)] = v * 4 + off
                return cc2
            lax.fori_loop(0, MACRO // LANES, oloop, 0)

        def zero_desc(j):
            lo = base + jnp.minimum(j * CHUNK, zlast)
            return pltpu.make_async_copy(zrow, acc.at[pl.ds(lo, CHUNK), :], zsem)

        def zero_cnt_desc(j):
            lo = base + jnp.minimum(j * CHUNK, zlast)
            return pltpu.make_async_copy(z128, cnt_acc.at[pl.ds(lo, CHUNK)], zsem)

        lax.fori_loop(0, nz, lambda j, cc: (zero_desc(j).start(), cc)[1], 0)

        @pl.when(p == 0)
        def _():
            lax.fori_loop(0, nz, lambda j, cc: (zero_cnt_desc(j).start(), cc)[1], 0)

        lax.fori_loop(0, nz, lambda j, cc: (zero_desc(j).wait(), cc)[1], 0)

        @pl.when(p == 0)
        def _():
            lax.fori_loop(0, nz, lambda j, cc: (zero_cnt_desc(j).wait(), cc)[1], 0)

        plsc.subcore_barrier()

        for d in idx_descs(0, 0):
            d.start()
        for d in idx_descs(0, 0):
            d.wait()
        for d in idx_descs(1, 1):
            d.start()
        bias_tgt(0)
        for d in gather_descs(0):
            d.start()

        def process(m, cur, oth):
            tgt_c, src_c, w_c, rows_c, scat_c = bufs[cur]

            for d in gather_descs(cur):
                d.wait()

            @pl.when(m >= 1)
            def _():
                for d in scat_descs(oth):
                    d.wait()

                @pl.when(p == 0)
                def _():
                    for d in cnt_descs(oth):
                        d.wait()

            @pl.when(m + 1 < nmacro)
            def _():
                for d in idx_descs(m + 1, oth):
                    d.wait()
                bias_tgt(oth)
                for d in gather_descs(oth):
                    d.start()

            def gloop(g, cc2):
                w_vec = w_c[pl.ds(g * LANES, LANES)]
                for e in range(LANES):
                    sp = _splat(w_vec, e)
                    q = g * LANES + e
                    a0 = rows_c[q, pl.ds(0, LANES)]
                    a1 = rows_c[q, pl.ds(LANES, LANES)]
                    rows_c[q, pl.ds(0, LANES)] = a0 * sp
                    rows_c[q, pl.ds(LANES, LANES)] = a1 * sp
                return cc2
            lax.fori_loop(0, MACRO // LANES, gloop, 0)
            for j in range(KROWS):
                for g2 in range(CHUNK // LANES):
                    scat_c[j, pl.ds(g2 * LANES, LANES)] = \
                        src_c[j, pl.ds(g2 * LANES, LANES)]

            for d in scat_descs(cur):
                d.start(add=True)

            @pl.when(p == 0)
            def _():
                for d in cnt_descs(cur):
                    d.start(add=True)

            @pl.when(m + 2 < nmacro)
            def _():
                for d in idx_descs(m + 2, cur):
                    d.start()

        def pair(mm, cc):
            m0 = 2 * mm
            process(m0, 0, 1)
            process(m0 + 1, 1, 0)
            return cc
        lax.fori_loop(0, nmacro // 2, pair, 0)

        for d in scat_descs(1):
            d.wait()

        @pl.when(p == 0)
        def _():
            for d in cnt_descs(1):
                d.wait()

        plsc.subcore_barrier()

        def wout_desc(j):
            lo = base + jnp.minimum(j * CHUNK, zlast)
            return pltpu.make_async_copy(
                acc.at[pl.ds(lo, CHUNK), :],
                aggr_out.at[r, pl.ds(lo, CHUNK), pl.ds(p * SLAB, SLAB)], zsem)

        def wout_cnt_desc(j):
            lo = base + jnp.minimum(j * CHUNK, zlast)
            return pltpu.make_async_copy(
                cnt_acc.at[pl.ds(lo, CHUNK)],
                cnt_out.at[r, pl.ds(lo, CHUNK)], zsem)

        lax.fori_loop(0, nz, lambda j, cc: (wout_desc(j).start(), cc)[1], 0)

        @pl.when(p == 0)
        def _():
            lax.fori_loop(0, nz, lambda j, cc: (wout_cnt_desc(j).start(), cc)[1], 0)

        lax.fori_loop(0, nz, lambda j, cc: (wout_desc(j).wait(), cc)[1], 0)

        @pl.when(p == 0)
        def _():
            lax.fori_loop(0, nz, lambda j, cc: (wout_cnt_desc(j).wait(), cc)[1], 0)
        return carry

    lax.fori_loop(0, NUNITS // NCORES, unit_body, 0)


def _make_sc_call(np_, acc_n, ept):
    mesh = plsc.VectorSubcoreMesh(
        core_axis_name="c", subcore_axis_name="s",
        num_cores=NCORES, num_subcores=NTILES)
    return pl.kernel(
        functools.partial(_sc_body, np_, acc_n, ept),
        out_type=[
            jax.ShapeDtypeStruct((4, np_, D), jnp.float32),
            jax.ShapeDtypeStruct((4, np_), jnp.float32),
        ],
        mesh=mesh,
        compiler_params=pltpu.CompilerParams(use_tc_tiling_on_sc=False),
        scratch_types=[
            pltpu.VMEM_SHARED((acc_n, SLAB), jnp.float32),
            pltpu.VMEM_SHARED((acc_n,), jnp.float32),
            pltpu.VMEM((MACRO,), jnp.int32),
            pltpu.VMEM((MACRO,), jnp.int32),
            pltpu.VMEM((KROWS, CHUNK), jnp.int32),
            pltpu.VMEM((KROWS, CHUNK), jnp.int32),
            pltpu.VMEM((MACRO,), jnp.float32),
            pltpu.VMEM((MACRO,), jnp.float32),
            pltpu.VMEM((MACRO, SLAB), jnp.float32),
            pltpu.VMEM((MACRO, SLAB), jnp.float32),
            pltpu.VMEM((KROWS, CHUNK), jnp.int32),
            pltpu.VMEM((KROWS, CHUNK), jnp.int32),
            pltpu.VMEM((CHUNK, SLAB), jnp.float32),
            pltpu.VMEM((CHUNK,), jnp.float32),
            pltpu.VMEM((CHUNK,), jnp.float32),
            pltpu.SemaphoreType.DMA,
            pltpu.SemaphoreType.DMA,
            pltpu.SemaphoreType.DMA,
            pltpu.SemaphoreType.DMA,
            pltpu.SemaphoreType.DMA,
        ],
    )


def _final_body(ag_ref, cnt_ref, xn_ref, u1_ref, u2_ref,
                wo1_ref, wo2_ref, bo_ref, out_ref):
    xn = xn_ref[...]
    zn = jnp.dot(xn, u2_ref[...], preferred_element_type=jnp.float32)
    aggs = []
    scores = []
    for r in range(4):
        cnt = jnp.maximum(cnt_ref[r], 1.0)
        a = ag_ref[r] / cnt[:, None]
        z = jnp.dot(a, u1_ref[...], preferred_element_type=jnp.float32) + zn
        z = jnp.where(z >= 0.0, z, 0.01 * z)
        aggs.append(a)
        scores.append(jnp.exp(z))
    ssum = scores[0] + scores[1] + scores[2] + scores[3]
    comb = aggs[0] * (scores[0] / ssum)
    for r in range(1, 4):
        comb = comb + aggs[r] * (scores[r] / ssum)
    h = (jnp.dot(xn, wo1_ref[...], preferred_element_type=jnp.float32)
         + jnp.dot(comb, wo2_ref[...], preferred_element_type=jnp.float32)
         + bo_ref[...])
    h = jnp.maximum(h, 0.0)
    nrm = jnp.sqrt(jnp.sum(h * h, axis=1, keepdims=True))
    out_ref[...] = h / jnp.maximum(nrm, 1e-12)


def _make_final_call(n, np_, bn):
    grid = (_cdiv(n, bn),)
    return pl.pallas_call(
        _final_body,
        grid=grid,
        in_specs=[
            pl.BlockSpec((4, bn, D), lambda i: (0, i, 0)),
            pl.BlockSpec((4, bn), lambda i: (0, i)),
            pl.BlockSpec((bn, D), lambda i: (i, 0)),
            pl.BlockSpec((D, 1), lambda i: (0, 0)),
            pl.BlockSpec((D, 1), lambda i: (0, 0)),
            pl.BlockSpec((D, D), lambda i: (0, 0)),
            pl.BlockSpec((D, D), lambda i: (0, 0)),
            pl.BlockSpec((1, D), lambda i: (0, 0)),
        ],
        out_specs=pl.BlockSpec((bn, D), lambda i: (i, 0)),
        out_shape=jax.ShapeDtypeStruct((n, D), jnp.float32),
        compiler_params=pltpu.CompilerParams(
            dimension_semantics=("arbitrary",),
        ),
    )


def kernel(x0, x1, x2, x3, e0, e1, e2, e3, w0, w1, w2, w3, x_node, num_node,
           W_a, b_a, W_p, b_p, W_t, b_t, W_c, b_c, u, W_out, b_out):
    n, d = x_node.shape
    assert d == D
    e = e0.shape[1]

    np_ = _cdiv(n, NTILES * 128) * NTILES * 128
    acc_n = _cdiv(n, NTILES * 8) * NTILES * 8
    ept = _cdiv(e, NTILES * 2 * MACRO) * 2 * MACRO
    epad = NTILES * ept

    ws_mat = jnp.stack([W_a, W_p, W_t, W_c])
    bs = jnp.stack([b_a, b_p, b_t, b_c])

    pad = epad - e
    ar = jnp.arange(pad, dtype=jnp.int32)
    pad_src = n + ar % (acc_n - n)
    pad_tgt = ar % 64
    srcs, tgts = [], []
    for er in (e0, e1, e2, e3):
        srcs.append(jnp.concatenate([er[0], pad_src]))
        tgts.append(jnp.concatenate([er[1], pad_tgt]))
    src_hbm = jnp.stack(srcs).reshape(4, 1, epad // CHUNK, CHUNK)
    tgt_hbm = jnp.stack(tgts)
    w_hbm = jnp.stack([
        jnp.concatenate([wr, jnp.zeros((pad,), jnp.float32)])
        for wr in (w0, w1, w2, w3)
    ])

    tables = _make_xt_call(np_, np_ // 16)(x0, x1, x2, x3, ws_mat, bs)
    table_flat = tables.reshape(4 * NSLAB * np_, SLAB)

    aggr16, cnts = _make_sc_call(np_, acc_n, ept)(
        table_flat, tgt_hbm, src_hbm, w_hbm)

    u1 = u[:D]
    u2 = u[D:]
    wo1 = W_out[:D]
    wo2 = W_out[D:]
    return _make_final_call(n, np_, np_ // 16)(
        aggr16, cnts, x_node, u1, u2, wo1, wo2, b_out.reshape(1, D))


# speedup vs baseline: 7.2644x; 1.1409x over previous
import functools

import jax
import jax.numpy as jnp
from jax import lax
from jax.experimental import pallas as pl
from jax.experimental.pallas import tpu as pltpu
from jax.experimental.pallas import tpu_sc as plsc

NCORES = 2
NTILES = 16
LANES = 16

D = 128
SLAB = 32
NSLAB = D // SLAB
NUNITS = 4 * NSLAB

CHUNK = 128
MACRO = CHUNK
DEPTH = 4


def _cdiv(a, b):
    return (a + b - 1) // b


def _splat(vec, e):
    return jnp.take_along_axis(vec, jnp.full((LANES,), e, jnp.int32), axis=0)


def _xt_body(x0r, x1r, x2r, x3r, w_ref, b_ref, out_ref):
    for k, xr in enumerate((x0r, x1r, x2r, x3r)):
        res = jnp.dot(xr[...], w_ref[k], preferred_element_type=jnp.float32)
        out_ref[k] = jnp.maximum(res + b_ref[k][None, :], 0.0)


def _make_xt_call(np_, bm):
    grid = (np_ // bm,)
    return pl.pallas_call(
        _xt_body,
        grid=grid,
        in_specs=[pl.BlockSpec((bm, D), lambda i: (i, 0))] * 4 + [
            pl.BlockSpec((4, D, D), lambda i: (0, 0, 0)),
            pl.BlockSpec((4, D), lambda i: (0, 0)),
        ],
        out_specs=pl.BlockSpec((4, bm, D), lambda i: (0, i, 0)),
        out_shape=jax.ShapeDtypeStruct((4, np_, D), jnp.float32),
        compiler_params=pltpu.CompilerParams(
            dimension_semantics=("arbitrary",),
        ),
    )


def _sc_body(np_, acc_n, ept, table, tgt_hbm, src_hbm, w_hbm,
             aggr_out, cnt_out, *scratch):
    (acc, cnt_acc) = scratch[0:2]
    tgt = scratch[2:6]
    src = scratch[6:10]
    w_v = scratch[10:14]
    rows = scratch[14:18]
    scat = scratch[18:20]
    (zrow, z128, ones128) = scratch[20:23]
    isem = scratch[23:27]
    gsem = scratch[27:31]
    ssem = scratch[31:33]
    csem = scratch[33:35]
    zsem = scratch[35]

    c = lax.axis_index("c")
    t = lax.axis_index("s")

    stripe = acc_n // NTILES
    nz = _cdiv(stripe, CHUNK)
    zlast = stripe - CHUNK
    rows_pt = ept // CHUNK
    nmacro = ept // MACRO

    def _init(i, carry):
        for g2 in range(SLAB // LANES):
            zrow[i, pl.ds(g2 * LANES, LANES)] = jnp.zeros((LANES,), jnp.float32)
        return carry
    lax.fori_loop(0, CHUNK, _init, 0)

    def _init1(i, carry):
        z128[pl.ds(i * LANES, LANES)] = jnp.zeros((LANES,), jnp.float32)
        ones128[pl.ds(i * LANES, LANES)] = jnp.ones((LANES,), jnp.float32)
        return carry
    lax.fori_loop(0, CHUNK // LANES, _init1, 0)

    def unit_body(i, carry):
        r = i // 2
        p = 2 * lax.rem(i, 2) + lax.rem(r + c, 2)
        u = 4 * r + p
        base = t * stripe
        off = 4 * r * np_ + p

        def idx_descs(m, s):
            ebase = t * ept + m * MACRO
            rowb = t * rows_pt + m
            return (
                pltpu.make_async_copy(
                    tgt_hbm.at[r, pl.ds(ebase, MACRO)], tgt[s], isem[s]),
                pltpu.make_async_copy(
                    w_hbm.at[r, pl.ds(ebase, MACRO)], w_v[s], isem[s]),
                pltpu.make_async_copy(
                    src_hbm.at[r, 0, pl.ds(rowb, 1), :], src[s], isem[s]),
            )

        def g_desc(s):
            return pltpu.make_async_copy(table.at[tgt[s]], rows[s], gsem[s])

        def s_desc(s, sp):
            return pltpu.make_async_copy(
                rows[s], acc.at[scat[sp].at[0]], ssem[sp])

        def c_desc(sp):
            return pltpu.make_async_copy(
                ones128, cnt_acc.at[scat[sp].at[0]], csem[sp])

        def bias_tgt(s):
            def oloop(g, cc2):
                v = tgt[s][pl.ds(g * LANES, LANES)]
                tgt[s][pl.ds(g * LANES, LANES)] = v * 4 + off
                return cc2
            lax.fori_loop(0, MACRO // LANES, oloop, 0)

        def zero_desc(j):
            lo = base + jnp.minimum(j * CHUNK, zlast)
            return pltpu.make_async_copy(zrow, acc.at[pl.ds(lo, CHUNK), :], zsem)

        def zero_cnt_desc(j):
            lo = base + jnp.minimum(j * CHUNK, zlast)
            return pltpu.make_async_copy(z128, cnt_acc.at[pl.ds(lo, CHUNK)], zsem)

        lax.fori_loop(0, nz, lambda j, cc: (zero_desc(j).start(), cc)[1], 0)

        @pl.when(p == 0)
        def _():
            lax.fori_loop(0, nz, lambda j, cc: (zero_cnt_desc(j).start(), cc)[1], 0)

        lax.fori_loop(0, nz, lambda j, cc: (zero_desc(j).wait(), cc)[1], 0)

        @pl.when(p == 0)
        def _():
            lax.fori_loop(0, nz, lambda j, cc: (zero_cnt_desc(j).wait(), cc)[1], 0)

        plsc.subcore_barrier()

        for s in range(DEPTH):
            for d in idx_descs(s, s):
                d.start()
        for s in range(2):
            for d in idx_descs(s, s):
                d.wait()
            bias_tgt(s)
            g_desc(s).start()

        def process(m, s):
            sp = s % 2
            so = (s + 2) % DEPTH

            g_desc(s).wait()

            @pl.when(m >= 2)
            def _():
                s_desc(so, sp).wait()

                @pl.when(p == 0)
                def _():
                    c_desc(sp).wait()

            @pl.when(m + 2 < nmacro)
            def _():
                for d in idx_descs(m + 2, so):
                    d.wait()
                bias_tgt(so)
                g_desc(so).start()

            for g2 in range(CHUNK // LANES):
                scat[sp][0, pl.ds(g2 * LANES, LANES)] = \
                    src[s][0, pl.ds(g2 * LANES, LANES)]

            @pl.when(m + DEPTH < nmacro)
            def _():
                for d in idx_descs(m + DEPTH, s):
                    d.start()

            def gloop(g, cc2):
                w_vec = w_v[s][pl.ds(g * LANES, LANES)]
                for e in range(LANES):
                    sp_ = _splat(w_vec, e)
                    q = g * LANES + e
                    a0 = rows[s][q, pl.ds(0, LANES)]
                    a1 = rows[s][q, pl.ds(LANES, LANES)]
                    rows[s][q, pl.ds(0, LANES)] = a0 * sp_
                    rows[s][q, pl.ds(LANES, LANES)] = a1 * sp_
                return cc2
            lax.fori_loop(0, MACRO // LANES, gloop, 0)

            s_desc(s, sp).start(add=True)

            @pl.when(p == 0)
            def _():
                c_desc(sp).start(add=True)

        def quad(q, cc):
            m0 = DEPTH * q
            for k in range(DEPTH):
                process(m0 + k, k)
            return cc
        lax.fori_loop(0, nmacro // DEPTH, quad, 0)

        s_desc((nmacro - 2) % DEPTH, 0).wait()
        s_desc((nmacro - 1) % DEPTH, 1).wait()

        @pl.when(p == 0)
        def _():
            c_desc(0).wait()
            c_desc(1).wait()

        plsc.subcore_barrier()

        def wout_desc(j):
            lo = base + jnp.minimum(j * CHUNK, zlast)
            return pltpu.make_async_copy(
                acc.at[pl.ds(lo, CHUNK), :],
                aggr_out.at[r, pl.ds(lo, CHUNK), pl.ds(p * SLAB, SLAB)], zsem)

        def wout_cnt_desc(j):
            lo = base + jnp.minimum(j * CHUNK, zlast)
            return pltpu.make_async_copy(
                cnt_acc.at[pl.ds(lo, CHUNK)],
                cnt_out.at[r, pl.ds(lo, CHUNK)], zsem)

        lax.fori_loop(0, nz, lambda j, cc: (wout_desc(j).start(), cc)[1], 0)

        @pl.when(p == 0)
        def _():
            lax.fori_loop(0, nz, lambda j, cc: (wout_cnt_desc(j).start(), cc)[1], 0)

        lax.fori_loop(0, nz, lambda j, cc: (wout_desc(j).wait(), cc)[1], 0)

        @pl.when(p == 0)
        def _():
            lax.fori_loop(0, nz, lambda j, cc: (wout_cnt_desc(j).wait(), cc)[1], 0)
        return carry

    lax.fori_loop(0, NUNITS // NCORES, unit_body, 0)


def _make_sc_call(np_, acc_n, ept):
    mesh = plsc.VectorSubcoreMesh(
        core_axis_name="c", subcore_axis_name="s",
        num_cores=NCORES, num_subcores=NTILES)
    return pl.kernel(
        functools.partial(_sc_body, np_, acc_n, ept),
        out_type=[
            jax.ShapeDtypeStruct((4, np_, D), jnp.float32),
            jax.ShapeDtypeStruct((4, np_), jnp.float32),
        ],
        mesh=mesh,
        compiler_params=pltpu.CompilerParams(use_tc_tiling_on_sc=False),
        scratch_types=(
            [
                pltpu.VMEM_SHARED((acc_n, SLAB), jnp.float32),
                pltpu.VMEM_SHARED((acc_n,), jnp.float32),
            ]
            + [pltpu.VMEM((MACRO,), jnp.int32) for _ in range(DEPTH)]
            + [pltpu.VMEM((1, CHUNK), jnp.int32) for _ in range(DEPTH)]
            + [pltpu.VMEM((MACRO,), jnp.float32) for _ in range(DEPTH)]
            + [pltpu.VMEM((MACRO, SLAB), jnp.float32) for _ in range(DEPTH)]
            + [pltpu.VMEM((1, CHUNK), jnp.int32) for _ in range(2)]
            + [
                pltpu.VMEM((CHUNK, SLAB), jnp.float32),
                pltpu.VMEM((CHUNK,), jnp.float32),
                pltpu.VMEM((CHUNK,), jnp.float32),
            ]
            + [pltpu.SemaphoreType.DMA for _ in range(DEPTH)]
            + [pltpu.SemaphoreType.DMA for _ in range(DEPTH)]
            + [pltpu.SemaphoreType.DMA for _ in range(2)]
            + [pltpu.SemaphoreType.DMA for _ in range(2)]
            + [pltpu.SemaphoreType.DMA]
        ),
    )


def _final_body(ag_ref, cnt_ref, xn_ref, u1_ref, u2_ref,
                wo1_ref, wo2_ref, bo_ref, out_ref):
    xn = xn_ref[...]
    zn = jnp.dot(xn, u2_ref[...], preferred_element_type=jnp.float32)
    aggs = []
    scores = []
    for r in range(4):
        cnt = jnp.maximum(cnt_ref[r], 1.0)
        a = ag_ref[r] / cnt[:, None]
        z = jnp.dot(a, u1_ref[...], preferred_element_type=jnp.float32) + zn
        z = jnp.where(z >= 0.0, z, 0.01 * z)
        aggs.append(a)
        scores.append(jnp.exp(z))
    ssum = scores[0] + scores[1] + scores[2] + scores[3]
    comb = aggs[0] * (scores[0] / ssum)
    for r in range(1, 4):
        comb = comb + aggs[r] * (scores[r] / ssum)
    h = (jnp.dot(xn, wo1_ref[...], preferred_element_type=jnp.float32)
         + jnp.dot(comb, wo2_ref[...], preferred_element_type=jnp.float32)
         + bo_ref[...])
    h = jnp.maximum(h, 0.0)
    nrm = jnp.sqrt(jnp.sum(h * h, axis=1, keepdims=True))
    out_ref[...] = h / jnp.maximum(nrm, 1e-12)


def _make_final_call(n, np_, bn):
    grid = (_cdiv(n, bn),)
    return pl.pallas_call(
        _final_body,
        grid=grid,
        in_specs=[
            pl.BlockSpec((4, bn, D), lambda i: (0, i, 0)),
            pl.BlockSpec((4, bn), lambda i: (0, i)),
            pl.BlockSpec((bn, D), lambda i: (i, 0)),
            pl.BlockSpec((D, 1), lambda i: (0, 0)),
            pl.BlockSpec((D, 1), lambda i: (0, 0)),
            pl.BlockSpec((D, D), lambda i: (0, 0)),
            pl.BlockSpec((D, D), lambda i: (0, 0)),
            pl.BlockSpec((1, D), lambda i: (0, 0)),
        ],
        out_specs=pl.BlockSpec((bn, D), lambda i: (i, 0)),
        out_shape=jax.ShapeDtypeStruct((n, D), jnp.float32),
        compiler_params=pltpu.CompilerParams(
            dimension_semantics=("arbitrary",),
        ),
    )


def kernel(x0, x1, x2, x3, e0, e1, e2, e3, w0, w1, w2, w3, x_node, num_node,
           W_a, b_a, W_p, b_p, W_t, b_t, W_c, b_c, u, W_out, b_out):
    n, d = x_node.shape
    assert d == D
    e = e0.shape[1]

    np_ = _cdiv(n, NTILES * 128) * NTILES * 128
    acc_n = _cdiv(n, NTILES * 8) * NTILES * 8
    ept = _cdiv(e, NTILES * DEPTH * MACRO) * DEPTH * MACRO
    epad = NTILES * ept

    ws_mat = jnp.stack([W_a, W_p, W_t, W_c])
    bs = jnp.stack([b_a, b_p, b_t, b_c])

    pad = epad - e
    ar = jnp.arange(pad, dtype=jnp.int32)
    pad_src = n + ar % (acc_n - n)
    pad_tgt = ar % 64
    srcs, tgts = [], []
    for er in (e0, e1, e2, e3):
        srcs.append(jnp.concatenate([er[0], pad_src]))
        tgts.append(jnp.concatenate([er[1], pad_tgt]))
    src_hbm = jnp.stack(srcs).reshape(4, 1, epad // CHUNK, CHUNK)
    tgt_hbm = jnp.stack(tgts)
    w_hbm = jnp.stack([
        jnp.concatenate([wr, jnp.zeros((pad,), jnp.float32)])
        for wr in (w0, w1, w2, w3)
    ])

    tables = _make_xt_call(np_, np_ // 16)(x0, x1, x2, x3, ws_mat, bs)
    table_flat = tables.reshape(4 * NSLAB * np_, SLAB)

    aggr4, cnts = _make_sc_call(np_, acc_n, ept)(
        table_flat, tgt_hbm, src_hbm, w_hbm)

    u1 = u[:D]
    u2 = u[D:]
    wo1 = W_out[:D]
    wo2 = W_out[D:]
    return _make_final_call(n, np_, np_ // 16)(
        aggr4, cnts, x_node, u1, u2, wo1, wo2, b_out.reshape(1, D))
